# Initial kernel scaffold; baseline (speedup 1.0000x reference)
import jax
import jax.numpy as jnp
from jax.experimental import pallas as pl


def kernel(logits):
    raise NotImplementedError("write your pallas kernel here")


# speedup vs baseline: 133.5507x; 133.5507x over previous
import jax
import jax.numpy as jnp
from jax.experimental import pallas as pl
from jax.experimental.pallas import tpu as pltpu

_TOP_P = 0.8
_N_ITERS = 28
_ROWS_PER_BLOCK = 8


def _topp_body(x_ref, o_ref):
    x = x_ref[...]
    m = jnp.max(x, axis=-1, keepdims=True)
    e = jnp.exp(x - m)
    z = jnp.sum(e, axis=-1, keepdims=True)
    target = _TOP_P * z

    def body(_, carry):
        lo, hi = carry
        mid = 0.5 * (lo + hi)
        mass = jnp.sum(jnp.where(e > mid, e, 0.0), axis=-1, keepdims=True)
        gt = mass > target
        lo = jnp.where(gt, mid, lo)
        hi = jnp.where(gt, hi, mid)
        return lo, hi

    lo0 = jnp.zeros_like(z)
    hi0 = jnp.ones_like(z)
    lo, _ = jax.lax.fori_loop(0, _N_ITERS, body, (lo0, hi0))

    mask = e > lo
    s = jnp.sum(jnp.where(mask, e, 0.0), axis=-1, keepdims=True)
    o_ref[...] = jnp.where(mask, e / s, 0.0)


def kernel(logits):
    b, v = logits.shape
    grid = b // _ROWS_PER_BLOCK
    return pl.pallas_call(
        _topp_body,
        grid=(grid,),
        in_specs=[pl.BlockSpec((_ROWS_PER_BLOCK, v), lambda i: (i, 0))],
        out_specs=pl.BlockSpec((_ROWS_PER_BLOCK, v), lambda i: (i, 0)),
        out_shape=jax.ShapeDtypeStruct((b, v), jnp.float32),
    )(logits)
